# Initial kernel scaffold; baseline (speedup 1.0000x reference)
import jax
import jax.numpy as jnp
from jax.experimental import pallas as pl


def kernel(x, W, b, idx1, idx2):
    raise NotImplementedError("write your pallas kernel here")


# speedup vs baseline: 1.7668x; 1.7668x over previous
import jax
import jax.numpy as jnp
from jax.experimental import pallas as pl

_BR = 1024


def _fused_rows(x_ref, a_ref, b_ref, o_ref):
    o_ref[...] = (
        jnp.dot(x_ref[...], a_ref[...], preferred_element_type=jnp.float32)
        + b_ref[...]
    )


def kernel(x, W, b, idx1, idx2):
    Bm, Dm = x.shape
    H = W.shape[0]
    cols2 = 2 * jnp.arange(H, dtype=jnp.int32)
    cols1 = cols2 + 1
    A = jnp.zeros((Dm, Dm), jnp.float32)
    A = A.at[idx2, cols2].add(1.0)
    A = A.at[idx1, cols1].add(1.0)
    A = A.at[idx1[:, None], cols2[None, :]].add(W.T)
    bfull = jnp.zeros((1, Dm), jnp.float32).at[0, cols2].set(b)

    return pl.pallas_call(
        _fused_rows,
        grid=(Bm // _BR,),
        in_specs=[
            pl.BlockSpec((_BR, Dm), lambda i: (i, 0)),
            pl.BlockSpec((Dm, Dm), lambda i: (0, 0)),
            pl.BlockSpec((1, Dm), lambda i: (0, 0)),
        ],
        out_specs=pl.BlockSpec((_BR, Dm), lambda i: (i, 0)),
        out_shape=jax.ShapeDtypeStruct((Bm, Dm), jnp.float32),
    )(x, A, bfull)


# speedup vs baseline: 1.7726x; 1.0033x over previous
import jax
import jax.numpy as jnp
from jax.experimental import pallas as pl
from jax.experimental.pallas import tpu as pltpu

_BR = 1024


def _fused_rows(x_ref, a_ref, b_ref, o_ref):
    o_ref[...] = (
        jnp.dot(x_ref[...], a_ref[...], preferred_element_type=jnp.float32)
        + b_ref[...]
    )


def kernel(x, W, b, idx1, idx2):
    Bm, Dm = x.shape
    H = W.shape[0]
    cols2 = 2 * jnp.arange(H, dtype=jnp.int32)
    cols1 = cols2 + 1
    A = jnp.zeros((Dm, Dm), jnp.float32)
    A = A.at[idx2, cols2].add(1.0)
    A = A.at[idx1, cols1].add(1.0)
    A = A.at[idx1[:, None], cols2[None, :]].add(W.T)
    bfull = jnp.zeros((1, Dm), jnp.float32).at[0, cols2].set(b)

    return pl.pallas_call(
        _fused_rows,
        grid=(Bm // _BR,),
        in_specs=[
            pl.BlockSpec((_BR, Dm), lambda i: (i, 0)),
            pl.BlockSpec((Dm, Dm), lambda i: (0, 0)),
            pl.BlockSpec((1, Dm), lambda i: (0, 0)),
        ],
        out_specs=pl.BlockSpec((_BR, Dm), lambda i: (i, 0)),
        out_shape=jax.ShapeDtypeStruct((Bm, Dm), jnp.float32),
        compiler_params=pltpu.CompilerParams(
            dimension_semantics=("parallel",),
        ),
    )(x, A, bfull)


# speedup vs baseline: 2.1265x; 1.1996x over previous
import jax
import jax.numpy as jnp
from jax.experimental import pallas as pl
from jax.experimental.pallas import tpu as pltpu

_BR = 4096


def _fused_rows(x_ref, a_ref, b_ref, o_ref):
    o_ref[...] = (
        jnp.dot(x_ref[...], a_ref[...], preferred_element_type=jnp.float32)
        + b_ref[...]
    )


def kernel(x, W, b, idx1, idx2):
    Bm, Dm = x.shape
    H = W.shape[0]
    cols2 = 2 * jnp.arange(H, dtype=jnp.int32)
    cols1 = cols2 + 1
    A = jnp.zeros((Dm, Dm), jnp.float32)
    A = A.at[idx2, cols2].add(1.0)
    A = A.at[idx1, cols1].add(1.0)
    A = A.at[idx1[:, None], cols2[None, :]].add(W.T)
    bfull = jnp.zeros((1, Dm), jnp.float32).at[0, cols2].set(b)

    return pl.pallas_call(
        _fused_rows,
        grid=(Bm // _BR,),
        in_specs=[
            pl.BlockSpec((_BR, Dm), lambda i: (i, 0)),
            pl.BlockSpec((Dm, Dm), lambda i: (0, 0)),
            pl.BlockSpec((1, Dm), lambda i: (0, 0)),
        ],
        out_specs=pl.BlockSpec((_BR, Dm), lambda i: (i, 0)),
        out_shape=jax.ShapeDtypeStruct((Bm, Dm), jnp.float32),
        compiler_params=pltpu.CompilerParams(
            dimension_semantics=("parallel",),
        ),
    )(x, A, bfull)


# speedup vs baseline: 2.1797x; 1.0250x over previous
import jax
import jax.numpy as jnp
from jax.experimental import pallas as pl
from jax.experimental.pallas import tpu as pltpu

_BR = 8192


def _fused_rows(x_ref, a_ref, b_ref, o_ref):
    o_ref[...] = (
        jnp.dot(x_ref[...], a_ref[...], preferred_element_type=jnp.float32)
        + b_ref[...]
    )


def kernel(x, W, b, idx1, idx2):
    Bm, Dm = x.shape
    H = W.shape[0]
    cols2 = 2 * jnp.arange(H, dtype=jnp.int32)
    cols1 = cols2 + 1
    A = jnp.zeros((Dm, Dm), jnp.float32)
    A = A.at[idx2, cols2].add(1.0)
    A = A.at[idx1, cols1].add(1.0)
    A = A.at[idx1[:, None], cols2[None, :]].add(W.T)
    bfull = jnp.zeros((1, Dm), jnp.float32).at[0, cols2].set(b)

    return pl.pallas_call(
        _fused_rows,
        grid=(Bm // _BR,),
        in_specs=[
            pl.BlockSpec((_BR, Dm), lambda i: (i, 0)),
            pl.BlockSpec((Dm, Dm), lambda i: (0, 0)),
            pl.BlockSpec((1, Dm), lambda i: (0, 0)),
        ],
        out_specs=pl.BlockSpec((_BR, Dm), lambda i: (i, 0)),
        out_shape=jax.ShapeDtypeStruct((Bm, Dm), jnp.float32),
        compiler_params=pltpu.CompilerParams(
            dimension_semantics=("parallel",),
        ),
    )(x, A, bfull)


# speedup vs baseline: 10.0553x; 4.6132x over previous
import jax
import jax.numpy as jnp
from jax.experimental import pallas as pl
from jax.experimental.pallas import tpu as pltpu

_BR = 8192


def _fused_rows(x_ref, a_ref, b_ref, o_ref):
    o_ref[...] = (
        jnp.dot(x_ref[...], a_ref[...], preferred_element_type=jnp.float32)
        + b_ref[...]
    )


def kernel(x, W, b, idx1, idx2):
    Bm, Dm = x.shape
    H = W.shape[0]
    ar = jnp.arange(Dm, dtype=jnp.int32)
    P1 = (idx1[:, None] == ar[None, :]).astype(jnp.float32)
    P2 = (idx2[:, None] == ar[None, :]).astype(jnp.float32)
    A_even = P2.T + P1.T @ W.T
    A_odd = P1.T
    A = jnp.stack([A_even, A_odd], axis=-1).reshape(Dm, Dm)
    bfull = jnp.stack([b, jnp.zeros_like(b)], axis=-1).reshape(1, Dm)

    return pl.pallas_call(
        _fused_rows,
        grid=(Bm // _BR,),
        in_specs=[
            pl.BlockSpec((_BR, Dm), lambda i: (i, 0)),
            pl.BlockSpec((Dm, Dm), lambda i: (0, 0)),
            pl.BlockSpec((1, Dm), lambda i: (0, 0)),
        ],
        out_specs=pl.BlockSpec((_BR, Dm), lambda i: (i, 0)),
        out_shape=jax.ShapeDtypeStruct((Bm, Dm), jnp.float32),
        compiler_params=pltpu.CompilerParams(
            dimension_semantics=("parallel",),
        ),
    )(x, A, bfull)


# speedup vs baseline: 10.5475x; 1.0489x over previous
import jax
import jax.numpy as jnp
from jax.experimental import pallas as pl
from jax.experimental.pallas import tpu as pltpu

_BR = 16384


def _fused_rows(x_ref, a_ref, b_ref, o_ref):
    o_ref[...] = (
        jnp.dot(x_ref[...], a_ref[...], preferred_element_type=jnp.float32)
        + b_ref[...]
    )


def kernel(x, W, b, idx1, idx2):
    Bm, Dm = x.shape
    H = W.shape[0]
    ar = jnp.arange(Dm, dtype=jnp.int32)
    P1 = (idx1[:, None] == ar[None, :]).astype(jnp.float32)
    P2 = (idx2[:, None] == ar[None, :]).astype(jnp.float32)
    A_even = P2.T + P1.T @ W.T
    A_odd = P1.T
    A = jnp.stack([A_even, A_odd], axis=-1).reshape(Dm, Dm)
    bfull = jnp.stack([b, jnp.zeros_like(b)], axis=-1).reshape(1, Dm)

    return pl.pallas_call(
        _fused_rows,
        grid=(Bm // _BR,),
        in_specs=[
            pl.BlockSpec((_BR, Dm), lambda i: (i, 0)),
            pl.BlockSpec((Dm, Dm), lambda i: (0, 0)),
            pl.BlockSpec((1, Dm), lambda i: (0, 0)),
        ],
        out_specs=pl.BlockSpec((_BR, Dm), lambda i: (i, 0)),
        out_shape=jax.ShapeDtypeStruct((Bm, Dm), jnp.float32),
        compiler_params=pltpu.CompilerParams(
            dimension_semantics=("parallel",),
        ),
    )(x, A, bfull)
